# baseline (speedup 1.0000x reference)
import jax
import jax.numpy as jnp
from jax import lax
from jax.experimental import pallas as pl
from jax.experimental.pallas import tpu as pltpu, tpu_sc as plsc

BATCH, SEQ, D = 4, 8192, 1024
NC, NS = 2, 16
NW = NC * NS
SEQ_PER_W = SEQ // NW
CH = 32
CHW = CH * D
NCH = SEQ_PER_W // CH


def _sc_body(x_hbm, emb_hbm, out_hbm, emb_v, xb):
    cid = lax.axis_index("c")
    sid = lax.axis_index("s")
    wid = sid * NC + cid
    seq0 = wid * SEQ_PER_W

    @pl.loop(0, NCH)
    def _chunk(c):
        off = (seq0 + c * CH) * D
        pltpu.sync_copy(emb_hbm.at[pl.ds(off, CHW)], emb_v)
        for b in range(BATCH):
            xoff = b * SEQ * D + off
            pltpu.sync_copy(x_hbm.at[pl.ds(xoff, CHW)], xb)

            @pl.loop(0, CHW, step=16, unroll=8)
            def _add(i):
                plsc.addupdate(xb.at[pl.ds(i, 16)], emb_v[pl.ds(i, 16)])

            pltpu.sync_copy(xb, out_hbm.at[pl.ds(xoff, CHW)])


def kernel(x, emb):
    xf = x.reshape(-1)
    ef = emb.reshape(-1)
    mesh = plsc.VectorSubcoreMesh(core_axis_name="c", subcore_axis_name="s")
    out = pl.kernel(
        _sc_body,
        out_type=jax.ShapeDtypeStruct((BATCH * SEQ * D,), jnp.float32),
        mesh=mesh,
        scratch_types=[
            pltpu.VMEM((CHW,), jnp.float32),
            pltpu.VMEM((CHW,), jnp.float32),
        ],
    )(xf, ef)
    return out.reshape(x.shape)


BLOCK_S = 2048


def _tc_body(x_ref, emb_ref, out_ref):
    out_ref[...] = x_ref[...] + emb_ref[...][None]


def _tc_kernel(x, emb):
    batch, seq_len, d_model = x.shape
    grid = (seq_len // BLOCK_S, batch)
    return pl.pallas_call(
        _tc_body,
        grid=grid,
        in_specs=[
            pl.BlockSpec((1, BLOCK_S, d_model), lambda s, b: (b, s, 0)),
            pl.BlockSpec((BLOCK_S, d_model), lambda s, b: (s, 0)),
        ],
        out_specs=pl.BlockSpec((1, BLOCK_S, d_model), lambda s, b: (b, s, 0)),
        out_shape=jax.ShapeDtypeStruct(x.shape, x.dtype),
    )(x, emb)


# speedup vs baseline: 1.1531x; 1.1531x over previous
import jax
import jax.numpy as jnp
from jax import lax
from jax.experimental import pallas as pl
from jax.experimental.pallas import tpu as pltpu, tpu_sc as plsc

BATCH, SEQ, D = 4, 8192, 1024
NC, NS = 2, 16
NW = NC * NS
SEQ_PER_W = SEQ // NW
CH = 32
CHW = CH * D
NCH = SEQ_PER_W // CH


def _sc_body(x_hbm, emb_hbm, out_hbm, emb_v, xb0, xb1, sin0, sin1, sout0, sout1):
    cid = lax.axis_index("c")
    sid = lax.axis_index("s")
    wid = sid * NC + cid
    seq0 = wid * SEQ_PER_W
    xbufs = (xb0, xb1)
    sins = (sin0, sin1)
    souts = (sout0, sout1)

    @pl.loop(0, NCH)
    def _chunk(c):
        off = (seq0 + c * CH) * D
        pltpu.sync_copy(emb_hbm.at[pl.ds(off, CHW)], emb_v)

        def xoff(b):
            return b * SEQ * D + off

        loads = {}
        stores = {}
        loads[0] = pltpu.async_copy(
            x_hbm.at[pl.ds(xoff(0), CHW)], xbufs[0], sins[0])
        for b in range(BATCH):
            k = b % 2
            loads[b].wait()
            if b + 1 < BATCH:
                if b - 1 >= 0:
                    stores[b - 1].wait()
                loads[b + 1] = pltpu.async_copy(
                    x_hbm.at[pl.ds(xoff(b + 1), CHW)],
                    xbufs[(b + 1) % 2], sins[(b + 1) % 2])

            xb = xbufs[k]

            @pl.loop(0, CHW, step=16, unroll=8)
            def _add(i):
                plsc.addupdate(xb.at[pl.ds(i, 16)], emb_v[pl.ds(i, 16)])

            stores[b] = pltpu.async_copy(
                xb, out_hbm.at[pl.ds(xoff(b), CHW)], souts[k])
        stores[BATCH - 2].wait()
        stores[BATCH - 1].wait()


def kernel(x, emb):
    xf = x.reshape(-1)
    ef = emb.reshape(-1)
    mesh = plsc.VectorSubcoreMesh(core_axis_name="c", subcore_axis_name="s")
    out = pl.kernel(
        _sc_body,
        out_type=jax.ShapeDtypeStruct((BATCH * SEQ * D,), jnp.float32),
        mesh=mesh,
        scratch_types=[
            pltpu.VMEM((CHW,), jnp.float32),
            pltpu.VMEM((CHW,), jnp.float32),
            pltpu.VMEM((CHW,), jnp.float32),
            pltpu.SemaphoreType.DMA,
            pltpu.SemaphoreType.DMA,
            pltpu.SemaphoreType.DMA,
            pltpu.SemaphoreType.DMA,
        ],
    )(xf, ef)
    return out.reshape(x.shape)


BLOCK_S = 2048


def _tc_body(x_ref, emb_ref, out_ref):
    out_ref[...] = x_ref[...] + emb_ref[...][None]


def _tc_kernel(x, emb):
    batch, seq_len, d_model = x.shape
    grid = (seq_len // BLOCK_S, batch)
    return pl.pallas_call(
        _tc_body,
        grid=grid,
        in_specs=[
            pl.BlockSpec((1, BLOCK_S, d_model), lambda s, b: (b, s, 0)),
            pl.BlockSpec((BLOCK_S, d_model), lambda s, b: (s, 0)),
        ],
        out_specs=pl.BlockSpec((1, BLOCK_S, d_model), lambda s, b: (b, s, 0)),
        out_shape=jax.ShapeDtypeStruct(x.shape, x.dtype),
    )(x, emb)


# speedup vs baseline: 1.3443x; 1.1658x over previous
import jax
import jax.numpy as jnp
from jax import lax
from jax.experimental import pallas as pl
from jax.experimental.pallas import tpu as pltpu, tpu_sc as plsc

BATCH, SEQ, D = 4, 8192, 1024
NC, NS = 2, 16
NW = NC * NS
SEQ_PER_W = SEQ // NW
CH = 32
CHW = CH * D
NCH = SEQ_PER_W // CH


def _sc_body(x_hbm, emb_hbm, out_hbm, emb_v, xb0, xb1, sin0, sin1, sout0, sout1):
    cid = lax.axis_index("c")
    sid = lax.axis_index("s")
    wid = sid * NC + cid
    seq0 = wid * SEQ_PER_W
    xbufs = (xb0, xb1)
    sins = (sin0, sin1)
    souts = (sout0, sout1)

    @pl.loop(0, NCH)
    def _chunk(c):
        row0 = seq0 + c * CH
        pltpu.sync_copy(emb_hbm.at[pl.ds(row0, CH)], emb_v)

        loads = {}
        stores = {}
        loads[0] = pltpu.async_copy(
            x_hbm.at[0, pl.ds(row0, CH)], xbufs[0], sins[0])
        for b in range(BATCH):
            k = b % 2
            loads[b].wait()
            if b + 1 < BATCH:
                if b - 1 >= 0:
                    stores[b - 1].wait()
                loads[b + 1] = pltpu.async_copy(
                    x_hbm.at[b + 1, pl.ds(row0, CH)],
                    xbufs[(b + 1) % 2], sins[(b + 1) % 2])

            xb = xbufs[k]

            @pl.loop(0, CH)
            def _row(r):
                @pl.loop(0, D, step=16, unroll=8)
                def _add(i):
                    plsc.addupdate(xb.at[r, pl.ds(i, 16)],
                                   emb_v[r, pl.ds(i, 16)])

            stores[b] = pltpu.async_copy(
                xb, out_hbm.at[b, pl.ds(row0, CH)], souts[k])
        stores[BATCH - 2].wait()
        stores[BATCH - 1].wait()


def kernel(x, emb):
    mesh = plsc.VectorSubcoreMesh(core_axis_name="c", subcore_axis_name="s")
    return pl.kernel(
        _sc_body,
        out_type=jax.ShapeDtypeStruct((BATCH, SEQ, D), jnp.float32),
        mesh=mesh,
        scratch_types=[
            pltpu.VMEM((CH, D), jnp.float32),
            pltpu.VMEM((CH, D), jnp.float32),
            pltpu.VMEM((CH, D), jnp.float32),
            pltpu.SemaphoreType.DMA,
            pltpu.SemaphoreType.DMA,
            pltpu.SemaphoreType.DMA,
            pltpu.SemaphoreType.DMA,
        ],
    )(x, emb)


BLOCK_S = 2048


def _tc_body(x_ref, emb_ref, out_ref):
    out_ref[...] = x_ref[...] + emb_ref[...][None]


def _tc_kernel(x, emb):
    batch, seq_len, d_model = x.shape
    grid = (seq_len // BLOCK_S, batch)
    return pl.pallas_call(
        _tc_body,
        grid=grid,
        in_specs=[
            pl.BlockSpec((1, BLOCK_S, d_model), lambda s, b: (b, s, 0)),
            pl.BlockSpec((BLOCK_S, d_model), lambda s, b: (s, 0)),
        ],
        out_specs=pl.BlockSpec((1, BLOCK_S, d_model), lambda s, b: (b, s, 0)),
        out_shape=jax.ShapeDtypeStruct(x.shape, x.dtype),
    )(x, emb)


# speedup vs baseline: 2.8645x; 2.1309x over previous
import jax
import jax.numpy as jnp
from jax import lax
from jax.experimental import pallas as pl
from jax.experimental.pallas import tpu as pltpu, tpu_sc as plsc

BATCH, SEQ, D = 4, 8192, 1024
NC, NS = 2, 16
NW = NC * NS
SEQ_PER_W = SEQ // NW
CH = 32
CHW = CH * D
NCH = SEQ_PER_W // CH


def _sc_body(x_hbm, emb_hbm, out_hbm, emb_v, xb0, xb1, sin0, sin1, sout0, sout1):
    cid = lax.axis_index("c")
    sid = lax.axis_index("s")
    wid = sid * NC + cid
    seq0 = wid * SEQ_PER_W
    xbufs = (xb0, xb1)
    sins = (sin0, sin1)
    souts = (sout0, sout1)

    @pl.loop(0, NCH)
    def _chunk(c):
        row0 = seq0 + c * CH
        pltpu.sync_copy(emb_hbm.at[pl.ds(row0, CH)], emb_v)

        loads = {}
        stores = {}
        loads[0] = pltpu.async_copy(
            x_hbm.at[0, pl.ds(row0, CH)], xbufs[0], sins[0])
        for b in range(BATCH):
            k = b % 2
            loads[b].wait()
            if b + 1 < BATCH:
                if b - 1 >= 0:
                    stores[b - 1].wait()
                loads[b + 1] = pltpu.async_copy(
                    x_hbm.at[b + 1, pl.ds(row0, CH)],
                    xbufs[(b + 1) % 2], sins[(b + 1) % 2])

            xb = xbufs[k]

            @pl.loop(0, CH)
            def _row(r):
                @plsc.parallel_loop(0, D, step=16, unroll=8)
                def _add(i):
                    plsc.addupdate(xb.at[r, pl.ds(i, 16)],
                                   emb_v[r, pl.ds(i, 16)])

            stores[b] = pltpu.async_copy(
                xb, out_hbm.at[b, pl.ds(row0, CH)], souts[k])
        stores[BATCH - 2].wait()
        stores[BATCH - 1].wait()


def kernel(x, emb):
    mesh = plsc.VectorSubcoreMesh(core_axis_name="c", subcore_axis_name="s")
    return pl.kernel(
        _sc_body,
        out_type=jax.ShapeDtypeStruct((BATCH, SEQ, D), jnp.float32),
        mesh=mesh,
        scratch_types=[
            pltpu.VMEM((CH, D), jnp.float32),
            pltpu.VMEM((CH, D), jnp.float32),
            pltpu.VMEM((CH, D), jnp.float32),
            pltpu.SemaphoreType.DMA,
            pltpu.SemaphoreType.DMA,
            pltpu.SemaphoreType.DMA,
            pltpu.SemaphoreType.DMA,
        ],
    )(x, emb)


BLOCK_S = 2048


def _tc_body(x_ref, emb_ref, out_ref):
    out_ref[...] = x_ref[...] + emb_ref[...][None]


def _tc_kernel(x, emb):
    batch, seq_len, d_model = x.shape
    grid = (seq_len // BLOCK_S, batch)
    return pl.pallas_call(
        _tc_body,
        grid=grid,
        in_specs=[
            pl.BlockSpec((1, BLOCK_S, d_model), lambda s, b: (b, s, 0)),
            pl.BlockSpec((BLOCK_S, d_model), lambda s, b: (s, 0)),
        ],
        out_specs=pl.BlockSpec((1, BLOCK_S, d_model), lambda s, b: (b, s, 0)),
        out_shape=jax.ShapeDtypeStruct(x.shape, x.dtype),
    )(x, emb)


# speedup vs baseline: 2.8661x; 1.0006x over previous
import jax
import jax.numpy as jnp
from jax import lax
from jax.experimental import pallas as pl
from jax.experimental.pallas import tpu as pltpu, tpu_sc as plsc

BATCH, SEQ, D = 4, 8192, 1024
NC, NS = 2, 16
NW = NC * NS
SEQ_PER_W = SEQ // NW
CH = 32
CHW = CH * D
NCH = SEQ_PER_W // CH


def _sc_body(x_hbm, emb_hbm, out_hbm, emb_v, xb0, xb1, sin0, sin1, sout0, sout1):
    cid = lax.axis_index("c")
    sid = lax.axis_index("s")
    wid = sid * NC + cid
    seq0 = wid * SEQ_PER_W
    xbufs = (xb0, xb1)
    sins = (sin0, sin1)
    souts = (sout0, sout1)

    @pl.loop(0, NCH)
    def _chunk(c):
        row0 = seq0 + c * CH
        pltpu.sync_copy(emb_hbm.at[pl.ds(row0, CH)], emb_v)

        loads = {}
        stores = {}
        loads[0] = pltpu.async_copy(
            x_hbm.at[0, pl.ds(row0, CH)], xbufs[0], sins[0])
        for b in range(BATCH):
            k = b % 2
            loads[b].wait()
            if b + 1 < BATCH:
                if b - 1 >= 0:
                    stores[b - 1].wait()
                loads[b + 1] = pltpu.async_copy(
                    x_hbm.at[b + 1, pl.ds(row0, CH)],
                    xbufs[(b + 1) % 2], sins[(b + 1) % 2])

            xb = xbufs[k]

            @plsc.parallel_loop(0, CH)
            def _row(r):
                @plsc.parallel_loop(0, D, step=16, unroll=8)
                def _add(i):
                    plsc.addupdate(xb.at[r, pl.ds(i, 16)],
                                   emb_v[r, pl.ds(i, 16)])

            stores[b] = pltpu.async_copy(
                xb, out_hbm.at[b, pl.ds(row0, CH)], souts[k])
        stores[BATCH - 2].wait()
        stores[BATCH - 1].wait()


def kernel(x, emb):
    mesh = plsc.VectorSubcoreMesh(core_axis_name="c", subcore_axis_name="s")
    return pl.kernel(
        _sc_body,
        out_type=jax.ShapeDtypeStruct((BATCH, SEQ, D), jnp.float32),
        mesh=mesh,
        scratch_types=[
            pltpu.VMEM((CH, D), jnp.float32),
            pltpu.VMEM((CH, D), jnp.float32),
            pltpu.VMEM((CH, D), jnp.float32),
            pltpu.SemaphoreType.DMA,
            pltpu.SemaphoreType.DMA,
            pltpu.SemaphoreType.DMA,
            pltpu.SemaphoreType.DMA,
        ],
    )(x, emb)


BLOCK_S = 2048


def _tc_body(x_ref, emb_ref, out_ref):
    out_ref[...] = x_ref[...] + emb_ref[...][None]


def _tc_kernel(x, emb):
    batch, seq_len, d_model = x.shape
    grid = (seq_len // BLOCK_S, batch)
    return pl.pallas_call(
        _tc_body,
        grid=grid,
        in_specs=[
            pl.BlockSpec((1, BLOCK_S, d_model), lambda s, b: (b, s, 0)),
            pl.BlockSpec((BLOCK_S, d_model), lambda s, b: (s, 0)),
        ],
        out_specs=pl.BlockSpec((1, BLOCK_S, d_model), lambda s, b: (b, s, 0)),
        out_shape=jax.ShapeDtypeStruct(x.shape, x.dtype),
    )(x, emb)


# speedup vs baseline: 2.8671x; 1.0003x over previous
import jax
import jax.numpy as jnp
from jax import lax
from jax.experimental import pallas as pl
from jax.experimental.pallas import tpu as pltpu, tpu_sc as plsc

BATCH, SEQ, D = 4, 8192, 1024
NC, NS = 2, 16
NW = NC * NS
SEQ_PER_W = SEQ // NW
CH = 32
CHW = CH * D
NCH = SEQ_PER_W // CH


def _sc_body(x_hbm, emb_hbm, out_hbm, emb_v, xb0, xb1, sin0, sin1, sout0, sout1):
    cid = lax.axis_index("c")
    sid = lax.axis_index("s")
    wid = sid * NC + cid
    seq0 = wid * SEQ_PER_W
    xbufs = (xb0, xb1)
    sins = (sin0, sin1)
    souts = (sout0, sout1)

    @pl.loop(0, NCH)
    def _chunk(c):
        row0 = seq0 + c * CH
        pltpu.sync_copy(emb_hbm.at[pl.ds(row0, CH)], emb_v)

        loads = {}
        stores = {}
        loads[0] = pltpu.async_copy(
            x_hbm.at[0, pl.ds(row0, CH)], xbufs[0], sins[0])
        for b in range(BATCH):
            k = b % 2
            loads[b].wait()
            if b + 1 < BATCH:
                if b - 1 >= 0:
                    stores[b - 1].wait()
                loads[b + 1] = pltpu.async_copy(
                    x_hbm.at[b + 1, pl.ds(row0, CH)],
                    xbufs[(b + 1) % 2], sins[(b + 1) % 2])

            xb = xbufs[k]

            @plsc.parallel_loop(0, CH)
            def _row(r):
                @plsc.parallel_loop(0, D, step=16, unroll=8)
                def _add(i):
                    plsc.addupdate(xb.at[r, pl.ds(i, 16)],
                                   emb_v[r, pl.ds(i, 16)])

            stores[b] = pltpu.async_copy(
                xb, out_hbm.at[b, pl.ds(row0, CH)], souts[k])
        stores[BATCH - 2].wait()
        stores[BATCH - 1].wait()


def kernel(x, emb):
    mesh = plsc.VectorSubcoreMesh(core_axis_name="c", subcore_axis_name="s")
    return pl.kernel(
        _sc_body,
        out_type=jax.ShapeDtypeStruct((BATCH, SEQ, D), jnp.float32),
        mesh=mesh,
        compiler_params=pltpu.CompilerParams(use_tc_tiling_on_sc=True),
        scratch_types=[
            pltpu.VMEM((CH, D), jnp.float32),
            pltpu.VMEM((CH, D), jnp.float32),
            pltpu.VMEM((CH, D), jnp.float32),
            pltpu.SemaphoreType.DMA,
            pltpu.SemaphoreType.DMA,
            pltpu.SemaphoreType.DMA,
            pltpu.SemaphoreType.DMA,
        ],
    )(x, emb)


BLOCK_S = 2048


def _tc_body(x_ref, emb_ref, out_ref):
    out_ref[...] = x_ref[...] + emb_ref[...][None]


def _tc_kernel(x, emb):
    batch, seq_len, d_model = x.shape
    grid = (seq_len // BLOCK_S, batch)
    return pl.pallas_call(
        _tc_body,
        grid=grid,
        in_specs=[
            pl.BlockSpec((1, BLOCK_S, d_model), lambda s, b: (b, s, 0)),
            pl.BlockSpec((BLOCK_S, d_model), lambda s, b: (s, 0)),
        ],
        out_specs=pl.BlockSpec((1, BLOCK_S, d_model), lambda s, b: (b, s, 0)),
        out_shape=jax.ShapeDtypeStruct(x.shape, x.dtype),
    )(x, emb)


# speedup vs baseline: 3.0787x; 1.0738x over previous
import jax
import jax.numpy as jnp
from jax import lax
from jax.experimental import pallas as pl
from jax.experimental.pallas import tpu as pltpu, tpu_sc as plsc

BATCH, SEQ, D = 4, 8192, 1024
NC, NS = 2, 16
NW = NC * NS
SEQ_PER_W = SEQ // NW
CH = 16
NCH = SEQ_PER_W // CH
NJOB = NCH * BATCH
XDEPTH = 4


def _sc_body(x_hbm, emb_hbm, out_hbm, *refs):
    ebufs = refs[0:2]
    xbufs = refs[2:2 + XDEPTH]
    ses = refs[6:8]
    sxs = refs[8:8 + XDEPTH]
    sos = refs[12:12 + XDEPTH]

    cid = lax.axis_index("c")
    sid = lax.axis_index("s")
    wid = sid * NC + cid
    seq0 = wid * SEQ_PER_W

    def row(c):
        return seq0 + c * CH

    eld = {}
    xld = {}
    xst = {}
    for c in range(2):
        eld[c] = pltpu.async_copy(
            emb_hbm.at[pl.ds(row(c), CH)], ebufs[c % 2], ses[c % 2])
    for j in range(XDEPTH - 1):
        xld[j] = pltpu.async_copy(
            x_hbm.at[j % BATCH, pl.ds(row(j // BATCH), CH)],
            xbufs[j % XDEPTH], sxs[j % XDEPTH])

    for j in range(NJOB):
        c, b = divmod(j, BATCH)
        k = j % XDEPTH
        if b == 0:
            eld[c].wait()
        xld[j].wait()
        jn = j + XDEPTH - 1
        if jn < NJOB:
            if j - 1 >= 0:
                xst[j - 1].wait()
            cn, bn = divmod(jn, BATCH)
            xld[jn] = pltpu.async_copy(
                x_hbm.at[bn, pl.ds(row(cn), CH)],
                xbufs[jn % XDEPTH], sxs[jn % XDEPTH])

        xb = xbufs[k]
        emb_v = ebufs[c % 2]

        @plsc.parallel_loop(0, CH)
        def _row(r):
            @plsc.parallel_loop(0, D, step=16, unroll=8)
            def _add(i):
                plsc.addupdate(xb.at[r, pl.ds(i, 16)],
                               emb_v[r, pl.ds(i, 16)])

        xst[j] = pltpu.async_copy(
            xb, out_hbm.at[b, pl.ds(row(c), CH)], sos[k])

        if b == BATCH - 1 and c + 2 < NCH:
            eld[c + 2] = pltpu.async_copy(
                emb_hbm.at[pl.ds(row(c + 2), CH)], ebufs[c % 2], ses[c % 2])

    for j in range(NJOB - XDEPTH, NJOB):
        xst[j].wait()


def kernel(x, emb):
    mesh = plsc.VectorSubcoreMesh(core_axis_name="c", subcore_axis_name="s")
    return pl.kernel(
        _sc_body,
        out_type=jax.ShapeDtypeStruct((BATCH, SEQ, D), jnp.float32),
        mesh=mesh,
        scratch_types=[
            pltpu.VMEM((CH, D), jnp.float32),
            pltpu.VMEM((CH, D), jnp.float32),
            pltpu.VMEM((CH, D), jnp.float32),
            pltpu.VMEM((CH, D), jnp.float32),
            pltpu.VMEM((CH, D), jnp.float32),
            pltpu.VMEM((CH, D), jnp.float32),
            pltpu.SemaphoreType.DMA,
            pltpu.SemaphoreType.DMA,
            pltpu.SemaphoreType.DMA,
            pltpu.SemaphoreType.DMA,
            pltpu.SemaphoreType.DMA,
            pltpu.SemaphoreType.DMA,
            pltpu.SemaphoreType.DMA,
            pltpu.SemaphoreType.DMA,
            pltpu.SemaphoreType.DMA,
            pltpu.SemaphoreType.DMA,
        ],
    )(x, emb)


BLOCK_S = 2048


def _tc_body(x_ref, emb_ref, out_ref):
    out_ref[...] = x_ref[...] + emb_ref[...][None]


def _tc_kernel(x, emb):
    batch, seq_len, d_model = x.shape
    grid = (seq_len // BLOCK_S, batch)
    return pl.pallas_call(
        _tc_body,
        grid=grid,
        in_specs=[
            pl.BlockSpec((1, BLOCK_S, d_model), lambda s, b: (b, s, 0)),
            pl.BlockSpec((BLOCK_S, d_model), lambda s, b: (s, 0)),
        ],
        out_specs=pl.BlockSpec((1, BLOCK_S, d_model), lambda s, b: (b, s, 0)),
        out_shape=jax.ShapeDtypeStruct(x.shape, x.dtype),
    )(x, emb)


# speedup vs baseline: 3.0903x; 1.0038x over previous
import jax
import jax.numpy as jnp
from jax import lax
from jax.experimental import pallas as pl
from jax.experimental.pallas import tpu as pltpu, tpu_sc as plsc

BATCH, SEQ, D = 4, 8192, 1024
NC, NS = 2, 16
NW = NC * NS
SEQ_PER_W = SEQ // NW
CH = 16
NCH = SEQ_PER_W // CH
NJOB = NCH * BATCH
XDEPTH = 5


def _sc_body(x_hbm, emb_hbm, out_hbm, *refs):
    ebufs = refs[0:2]
    xbufs = refs[2:2 + XDEPTH]
    ses = refs[2 + XDEPTH:4 + XDEPTH]
    sxs = refs[4 + XDEPTH:4 + 2 * XDEPTH]
    sos = refs[4 + 2 * XDEPTH:4 + 3 * XDEPTH]

    cid = lax.axis_index("c")
    sid = lax.axis_index("s")
    wid = sid * NC + cid
    seq0 = wid * SEQ_PER_W

    def row(c):
        return seq0 + c * CH

    eld = {}
    xld = {}
    xst = {}
    for c in range(2):
        eld[c] = pltpu.async_copy(
            emb_hbm.at[pl.ds(row(c), CH)], ebufs[c % 2], ses[c % 2])
    for j in range(XDEPTH - 1):
        xld[j] = pltpu.async_copy(
            x_hbm.at[j % BATCH, pl.ds(row(j // BATCH), CH)],
            xbufs[j % XDEPTH], sxs[j % XDEPTH])

    for j in range(NJOB):
        c, b = divmod(j, BATCH)
        k = j % XDEPTH
        if b == 0:
            eld[c].wait()
        xld[j].wait()
        jn = j + XDEPTH - 1
        if jn < NJOB:
            if j - 1 >= 0:
                xst[j - 1].wait()
            cn, bn = divmod(jn, BATCH)
            xld[jn] = pltpu.async_copy(
                x_hbm.at[bn, pl.ds(row(cn), CH)],
                xbufs[jn % XDEPTH], sxs[jn % XDEPTH])

        xb = xbufs[k]
        emb_v = ebufs[c % 2]

        @plsc.parallel_loop(0, CH)
        def _row(r):
            @plsc.parallel_loop(0, D, step=16, unroll=8)
            def _add(i):
                plsc.addupdate(xb.at[r, pl.ds(i, 16)],
                               emb_v[r, pl.ds(i, 16)])

        xst[j] = pltpu.async_copy(
            xb, out_hbm.at[b, pl.ds(row(c), CH)], sos[k])

        if b == BATCH - 1 and c + 2 < NCH:
            eld[c + 2] = pltpu.async_copy(
                emb_hbm.at[pl.ds(row(c + 2), CH)], ebufs[c % 2], ses[c % 2])

    for j in range(NJOB - XDEPTH, NJOB):
        xst[j].wait()


def kernel(x, emb):
    mesh = plsc.VectorSubcoreMesh(core_axis_name="c", subcore_axis_name="s")
    return pl.kernel(
        _sc_body,
        out_type=jax.ShapeDtypeStruct((BATCH, SEQ, D), jnp.float32),
        mesh=mesh,
        scratch_types=(
            [pltpu.VMEM((CH, D), jnp.float32)] * (2 + XDEPTH)
            + [pltpu.SemaphoreType.DMA] * (2 + 2 * XDEPTH)
        ),
    )(x, emb)


BLOCK_S = 2048


def _tc_body(x_ref, emb_ref, out_ref):
    out_ref[...] = x_ref[...] + emb_ref[...][None]


def _tc_kernel(x, emb):
    batch, seq_len, d_model = x.shape
    grid = (seq_len // BLOCK_S, batch)
    return pl.pallas_call(
        _tc_body,
        grid=grid,
        in_specs=[
            pl.BlockSpec((1, BLOCK_S, d_model), lambda s, b: (b, s, 0)),
            pl.BlockSpec((BLOCK_S, d_model), lambda s, b: (s, 0)),
        ],
        out_specs=pl.BlockSpec((1, BLOCK_S, d_model), lambda s, b: (b, s, 0)),
        out_shape=jax.ShapeDtypeStruct(x.shape, x.dtype),
    )(x, emb)


# speedup vs baseline: 3.2473x; 1.0508x over previous
import jax
import jax.numpy as jnp
from jax import lax
from jax.experimental import pallas as pl
from jax.experimental.pallas import tpu as pltpu, tpu_sc as plsc

BATCH, SEQ, D = 4, 8192, 1024
NC, NS = 2, 16
NW = NC * NS
SEQ_PER_W = SEQ // NW
CH = 8
NCH = SEQ_PER_W // CH
XDEPTH = 3


def _sc_body(x_hbm, emb_hbm, out_hbm, *refs):
    ebufs = refs[0:2]
    xbufs = refs[2:2 + XDEPTH]
    ses = refs[2 + XDEPTH:4 + XDEPTH]
    sxs = refs[4 + XDEPTH:4 + 2 * XDEPTH]
    sos = refs[4 + 2 * XDEPTH:4 + 3 * XDEPTH]

    cid = lax.axis_index("c")
    sid = lax.axis_index("s")
    wid = sid * NC + cid
    seq0 = wid * SEQ_PER_W

    def row(c):
        return seq0 + c * CH

    eld = {}
    xld = {}
    xst = {}
    for c in range(2):
        eld[c] = pltpu.async_copy(
            emb_hbm.at[pl.ds(row(c), CH)], ebufs[c % 2], ses[c % 2])
    for c in range(XDEPTH - 1):
        xld[c] = pltpu.async_copy(
            x_hbm.at[:, pl.ds(row(c), CH)],
            xbufs[c % XDEPTH], sxs[c % XDEPTH])

    for c in range(NCH):
        k = c % XDEPTH
        eld[c].wait()
        xld[c].wait()
        cn = c + XDEPTH - 1
        if cn < NCH:
            if c - 1 >= 0:
                xst[c - 1].wait()
            xld[cn] = pltpu.async_copy(
                x_hbm.at[:, pl.ds(row(cn), CH)],
                xbufs[cn % XDEPTH], sxs[cn % XDEPTH])

        xb = xbufs[k]
        emb_v = ebufs[c % 2]

        @plsc.parallel_loop(0, BATCH)
        def _batch(b):
            @plsc.parallel_loop(0, CH)
            def _row(r):
                @plsc.parallel_loop(0, D, step=16, unroll=8)
                def _add(i):
                    plsc.addupdate(xb.at[b, r, pl.ds(i, 16)],
                                   emb_v[r, pl.ds(i, 16)])

        xst[c] = pltpu.async_copy(
            xb, out_hbm.at[:, pl.ds(row(c), CH)], sos[k])

        if c + 2 < NCH:
            eld[c + 2] = pltpu.async_copy(
                emb_hbm.at[pl.ds(row(c + 2), CH)], ebufs[c % 2], ses[c % 2])

    for c in range(NCH - XDEPTH, NCH):
        xst[c].wait()


def kernel(x, emb):
    mesh = plsc.VectorSubcoreMesh(core_axis_name="c", subcore_axis_name="s")
    return pl.kernel(
        _sc_body,
        out_type=jax.ShapeDtypeStruct((BATCH, SEQ, D), jnp.float32),
        mesh=mesh,
        scratch_types=(
            [pltpu.VMEM((CH, D), jnp.float32)] * 2
            + [pltpu.VMEM((BATCH, CH, D), jnp.float32)] * XDEPTH
            + [pltpu.SemaphoreType.DMA] * (2 + 2 * XDEPTH)
        ),
    )(x, emb)
